# baseline (speedup 1.0000x reference)
import functools

import jax
import jax.numpy as jnp
import numpy as np
from jax import lax
from jax.experimental import pallas as pl
from jax.experimental.pallas import tpu as pltpu
from jax.experimental.pallas import tpu_sc as plsc

N_USER = 60000
N_ITEM = 40000
N = N_USER + N_ITEM
E_PAIRS = 800000
D = 32
BATCH = 1024

NC = 2
NS = 16
L = 16

CHUNK = 104
NBUF = 2
NGROUPS = 241
NBUF_H = 2
NGROUPS_H = 241
CHUNKS_PER_SUB = NBUF * NGROUPS
EDGES_PER_SUB = CHUNKS_PER_SUB * CHUNK
HALF_PAD = EDGES_PER_SUB * NS
ACC_ROWS = 60032
DUMP_ROW = 60000
ZROWS_PER_SUB = ACC_ROWS // NS
U_SPAN = 3752
U_LAST = N_USER - 15 * U_SPAN
I_SPAN = 2504
I_LAST = N_ITEM - 15 * I_SPAN
HIST_W = 16

_mesh = plsc.VectorSubcoreMesh(
    core_axis_name="c", subcore_axis_name="s", num_cores=NC, num_subcores=NS)
_sc_params = pltpu.CompilerParams(use_tc_tiling_on_sc=False,
                                  needs_layout_passes=False)


def _writeout(acc_sh, out, c, s):
    @pl.when(jnp.logical_and(c == 0, s < 15))
    def _():
        pltpu.sync_copy(acc_sh.at[pl.ds(s * U_SPAN, U_SPAN)],
                        out.at[pl.ds(s * U_SPAN, U_SPAN)])

    @pl.when(jnp.logical_and(c == 0, s == 15))
    def _():
        pltpu.sync_copy(acc_sh.at[pl.ds(15 * U_SPAN, U_LAST)],
                        out.at[pl.ds(15 * U_SPAN, U_LAST)])

    @pl.when(jnp.logical_and(c == 1, s < 15))
    def _():
        pltpu.sync_copy(acc_sh.at[pl.ds(s * I_SPAN, I_SPAN)],
                        out.at[pl.ds(N_USER + s * I_SPAN, I_SPAN)])

    @pl.when(jnp.logical_and(c == 1, s == 15))
    def _():
        pltpu.sync_copy(acc_sh.at[pl.ds(15 * I_SPAN, I_LAST)],
                        out.at[pl.ds(N_USER + 15 * I_SPAN, I_LAST)])


def _hist_body(rowp, zeros16, deg, acc_sh, basis, *rest):
    ridx = rest[0:NBUF_H]
    sem_i = rest[NBUF_H:2 * NBUF_H]
    sem_s = rest[2 * NBUF_H:3 * NBUF_H]
    c = lax.axis_index("c")
    s = lax.axis_index("s")
    pltpu.sync_copy(zeros16.at[pl.ds(s * ZROWS_PER_SUB, ZROWS_PER_SUB)],
                    acc_sh.at[pl.ds(s * ZROWS_PER_SUB, ZROWS_PER_SUB)])
    e0 = jnp.where(lax.iota(jnp.int32, L) == 0,
                   jnp.float32(1), jnp.float32(0))

    @pl.loop(0, CHUNK)
    def _(i):
        basis[i, :] = e0

    plsc.subcore_barrier()
    base = c * HALF_PAD + s * EDGES_PER_SUB

    @pl.loop(0, NGROUPS_H)
    def _(grp):
        cbase = base + grp * (NBUF_H * CHUNK)
        descs = []
        for b in range(NBUF_H):
            @pl.when(grp > 0)
            def _(b=b):
                pltpu.make_async_copy(
                    zeros16.at[pl.ds(0, CHUNK)], basis, sem_s[b]).wait()
            descs.append(pltpu.async_copy(
                rowp.at[pl.ds(cbase + b * CHUNK, CHUNK)], ridx[b], sem_i[b]))
        for b in range(NBUF_H):
            descs[b].wait()
            pltpu.async_copy(basis, acc_sh.at[ridx[b]], sem_s[b], add=True)

    for b in range(NBUF_H):
        pltpu.make_async_copy(
            zeros16.at[pl.ds(0, CHUNK)], basis, sem_s[b]).wait()

    plsc.subcore_barrier()
    _writeout(acc_sh, deg, c, s)


_hist_kernel = pl.kernel(
    _hist_body,
    out_type=jax.ShapeDtypeStruct((N, HIST_W), jnp.float32),
    mesh=_mesh,
    scratch_types=(
        [pltpu.VMEM_SHARED((ACC_ROWS, HIST_W), jnp.float32)]
        + [pltpu.VMEM((CHUNK, HIST_W), jnp.float32)]
        + [pltpu.VMEM((CHUNK,), jnp.int32)] * NBUF_H
        + [pltpu.SemaphoreType.DMA] * (2 * NBUF_H)
    ),
    compiler_params=_sc_params,
)


def _spmm_body(x, rowp, colp, zeros32, acc, acc_sh, *rest):
    ridx = rest[0:NBUF]
    cidx = rest[NBUF:2 * NBUF]
    rbuf = rest[2 * NBUF:3 * NBUF]
    sbuf = rest[3 * NBUF:4 * NBUF]
    sem_i = rest[4 * NBUF:5 * NBUF]
    sem_g = rest[5 * NBUF:6 * NBUF]
    sem_s = rest[6 * NBUF:7 * NBUF]
    c = lax.axis_index("c")
    s = lax.axis_index("s")
    pltpu.sync_copy(zeros32.at[pl.ds(s * ZROWS_PER_SUB, ZROWS_PER_SUB)],
                    acc_sh.at[pl.ds(s * ZROWS_PER_SUB, ZROWS_PER_SUB)])
    plsc.subcore_barrier()
    base = c * HALF_PAD + s * EDGES_PER_SUB

    @pl.loop(0, NGROUPS)
    def _(grp):
        cbase = base + grp * (NBUF * CHUNK)
        descs = []
        for b in range(NBUF):
            @pl.when(grp > 0)
            def _(b=b):
                pltpu.make_async_copy(
                    zeros32.at[pl.ds(0, CHUNK)], sbuf[b], sem_s[b]).wait()
            di = pltpu.async_copy(
                rowp.at[pl.ds(cbase + b * CHUNK, CHUNK)], ridx[b], sem_i[b])
            dj = pltpu.async_copy(
                colp.at[pl.ds(cbase + b * CHUNK, CHUNK)], cidx[b], sem_i[b])
            descs.append((di, dj))
        for b in range(NBUF):
            descs[b][0].wait()
            descs[b][1].wait()
            pltpu.async_copy(x.at[cidx[b]], rbuf[b], sem_g[b])
        for b in range(NBUF):
            pltpu.make_async_copy(
                x.at[pl.ds(0, CHUNK)], rbuf[b], sem_g[b]).wait()

            @pl.loop(0, CHUNK)
            def _(r, b=b):
                lo, hi = plsc.unpack(rbuf[b][r, :],
                                     format=plsc.PackFormat.INTERLEAVED)
                sbuf[b][r, 0:L] = lo
                sbuf[b][r, L:2 * L] = hi

            pltpu.async_copy(sbuf[b], acc_sh.at[ridx[b]], sem_s[b], add=True)

    for b in range(NBUF):
        pltpu.make_async_copy(
            zeros32.at[pl.ds(0, CHUNK)], sbuf[b], sem_s[b]).wait()

    plsc.subcore_barrier()
    _writeout(acc_sh, acc, c, s)


_spmm_kernel = pl.kernel(
    _spmm_body,
    out_type=jax.ShapeDtypeStruct((N, D), jnp.float32),
    mesh=_mesh,
    scratch_types=(
        [pltpu.VMEM_SHARED((ACC_ROWS, D), jnp.float32)]
        + [pltpu.VMEM((CHUNK,), jnp.int32)] * (2 * NBUF)
        + [pltpu.VMEM((CHUNK, D), jnp.bfloat16)] * NBUF
        + [pltpu.VMEM((CHUNK, D), jnp.float32)] * NBUF
        + [pltpu.SemaphoreType.DMA] * (3 * NBUF)
    ),
    compiler_params=_sc_params,
)

ROWS_PER_GW = BATCH // (NC * NS)


def _bgather_body(t0, t1, t2, t3, deg, bidx, o0, o1, o2, o3, odeg,
                  ibuf, bbuf, rbuf, dbuf, dma):
    c = lax.axis_index("c")
    s = lax.axis_index("s")
    w = s * NC + c
    base = w * (3 * BATCH // (NC * NS))
    span = 3 * BATCH // (NC * NS)
    pltpu.sync_copy(bidx.at[pl.ds(base, span)], ibuf)
    for t, out in ((t0, o0), (t1, o1), (t2, o2)):
        pltpu.async_copy(t.at[ibuf], bbuf, dma).wait()
        pltpu.sync_copy(bbuf, out.at[pl.ds(base, span)])
    pltpu.async_copy(t3.at[ibuf], rbuf, dma).wait()
    pltpu.sync_copy(rbuf, o3.at[pl.ds(base, span)])
    pltpu.async_copy(deg.at[ibuf], dbuf, dma).wait()
    pltpu.sync_copy(dbuf, odeg.at[pl.ds(base, span)])


_bgather_kernel = pl.kernel(
    _bgather_body,
    out_type=[jax.ShapeDtypeStruct((3 * BATCH, D), jnp.bfloat16)] * 3
    + [jax.ShapeDtypeStruct((3 * BATCH, D), jnp.float32)]
    + [jax.ShapeDtypeStruct((3 * BATCH, HIST_W), jnp.float32)],
    mesh=_mesh,
    scratch_types=[
        pltpu.VMEM((3 * BATCH // (NC * NS),), jnp.int32),
        pltpu.VMEM((3 * BATCH // (NC * NS), D), jnp.bfloat16),
        pltpu.VMEM((3 * BATCH // (NC * NS), D), jnp.float32),
        pltpu.VMEM((3 * BATCH // (NC * NS), HIST_W), jnp.float32),
        pltpu.SemaphoreType.DMA,
    ],
    compiler_params=_sc_params,
)


BR = 5000
GRID = N // BR


def _prep_body(deg_ref, ego_ref, p_ref, x_ref):
    dinv = lax.rsqrt(jnp.maximum(deg_ref[:, :1], 1.0))
    xw = jnp.dot(ego_ref[...] * dinv, p_ref[...],
                 preferred_element_type=jnp.float32)
    x_ref[...] = xw.astype(jnp.bfloat16)


_prep_call = pl.pallas_call(
    _prep_body,
    grid=(GRID,),
    in_specs=[
        pl.BlockSpec((BR, HIST_W), lambda i: (i, 0)),
        pl.BlockSpec((BR, D), lambda i: (i, 0)),
        pl.BlockSpec((D, D), lambda i: (0, 0)),
    ],
    out_specs=pl.BlockSpec((BR, D), lambda i: (i, 0)),
    out_shape=jax.ShapeDtypeStruct((N, D), jnp.bfloat16),
)


def _dense_body(acc_ref, ego_ref, deg_ref, wg_ref, bg_ref, wb_ref, bb_ref,
                p_ref, h_ref, xn_ref):
    dinv = lax.rsqrt(jnp.maximum(deg_ref[:, :1], 1.0))
    side = acc_ref[...] * dinv
    s_emb = jnp.dot(side, wg_ref[...],
                    preferred_element_type=jnp.float32) + bg_ref[...]
    b_emb = jnp.dot(ego_ref[...] * side, wb_ref[...],
                    preferred_element_type=jnp.float32) + bb_ref[...]
    z = s_emb + b_emb
    h = jnp.where(z >= 0, z, 0.2 * z)
    h_ref[...] = h
    xw = jnp.dot(h * dinv, p_ref[...], preferred_element_type=jnp.float32)
    xn_ref[...] = xw.astype(jnp.bfloat16)


_dense_call = pl.pallas_call(
    _dense_body,
    grid=(GRID,),
    in_specs=[
        pl.BlockSpec((BR, D), lambda i: (i, 0)),
        pl.BlockSpec((BR, D), lambda i: (i, 0)),
        pl.BlockSpec((BR, HIST_W), lambda i: (i, 0)),
        pl.BlockSpec((D, D), lambda i: (0, 0)),
        pl.BlockSpec((1, D), lambda i: (0, 0)),
        pl.BlockSpec((D, D), lambda i: (0, 0)),
        pl.BlockSpec((1, D), lambda i: (0, 0)),
        pl.BlockSpec((D, D), lambda i: (0, 0)),
    ],
    out_specs=[pl.BlockSpec((BR, D), lambda i: (i, 0)),
               pl.BlockSpec((BR, D), lambda i: (i, 0))],
    out_shape=[jax.ShapeDtypeStruct((N, D), jnp.float32),
               jax.ShapeDtypeStruct((N, D), jnp.bfloat16)],
)


def _finish_body(x0r, x1r, x2r, acc3r, degr, wg_ref, bg_ref, wb_ref, bb_ref,
                 pt_ref, ug, pg, ng):
    pt = pt_ref[...]
    x0v = jnp.dot(x0r[...].astype(jnp.float32), pt,
                  preferred_element_type=jnp.float32)
    x1v = jnp.dot(x1r[...].astype(jnp.float32), pt,
                  preferred_element_type=jnp.float32)
    x2v = jnp.dot(x2r[...].astype(jnp.float32), pt,
                  preferred_element_type=jnp.float32)
    dall = jnp.maximum(degr[:, :1], 1.0)
    scale_all = jnp.sqrt(dall)
    dinv_all = lax.rsqrt(dall)
    side3 = acc3r[...] * dinv_all
    ego2r = x2v * scale_all
    z = (jnp.dot(side3, wg_ref[...], preferred_element_type=jnp.float32)
         + bg_ref[...]
         + jnp.dot(ego2r * side3, wb_ref[...],
                   preferred_element_type=jnp.float32) + bb_ref[...])
    h3 = jnp.where(z >= 0, z, 0.2 * z)
    for b, out in enumerate((ug, pg, ng)):
        sl = slice(b * BATCH, (b + 1) * BATCH)
        out[:, 0:D] = x0v[sl, :] * scale_all[sl, :]
        for t, v in enumerate((x1v[sl, :], x2v[sl, :], h3[sl, :])):
            nrm = jnp.maximum(
                jnp.sqrt(jnp.sum(v * v, axis=1, keepdims=True)), 1e-12)
            out[:, (t + 1) * D:(t + 2) * D] = v / nrm


_finish_call = pl.pallas_call(
    _finish_body,
    grid=(1,),
    in_specs=[pl.BlockSpec((3 * BATCH, D), lambda i: (0, 0))] * 4
    + [pl.BlockSpec((3 * BATCH, HIST_W), lambda i: (0, 0))]
    + [pl.BlockSpec((D, D), lambda i: (0, 0)),
       pl.BlockSpec((1, D), lambda i: (0, 0)),
       pl.BlockSpec((D, D), lambda i: (0, 0)),
       pl.BlockSpec((1, D), lambda i: (0, 0)),
       pl.BlockSpec((D, D), lambda i: (0, 0))],
    out_specs=[pl.BlockSpec((BATCH, 4 * D), lambda i: (0, 0))] * 3,
    out_shape=[jax.ShapeDtypeStruct((BATCH, 4 * D), jnp.float32)] * 3,
)


def kernel(users, pos_items, neg_items, edge_index, adj_values, user_emb,
           item_emb, W_gc_0, b_gc_0, W_bi_0, b_bi_0, W_gc_1, b_gc_1, W_bi_1,
           b_bi_1, W_gc_2, b_gc_2, W_bi_2, b_bi_2):
    del adj_values
    row = edge_index[0].astype(jnp.int32)
    col = edge_index[1].astype(jnp.int32)
    pad_n = HALF_PAD - E_PAIRS
    pad_row = jnp.full((pad_n,), DUMP_ROW, jnp.int32)
    pad_col = jnp.zeros((pad_n,), jnp.int32)
    rowp = jnp.concatenate(
        [row[:E_PAIRS], pad_row, row[E_PAIRS:] - N_USER, pad_row])
    colp = jnp.concatenate([col[:E_PAIRS], pad_col, col[E_PAIRS:], pad_col])

    zeros16 = jnp.zeros((ACC_ROWS, HIST_W), jnp.float32)
    zeros32 = jnp.zeros((ACC_ROWS, D), jnp.float32)

    ego = jnp.concatenate([user_emb, item_emb], axis=0)
    perm = np.zeros((D, D), np.float32)
    for j in range(D // 2):
        perm[j, 2 * j] = 1.0
        perm[D // 2 + j, 2 * j + 1] = 1.0
    p_mat = jnp.asarray(perm)
    pt_mat = jnp.asarray(perm.T)
    deg = _hist_kernel(rowp, zeros16)
    x = _prep_call(deg, ego, p_mat)

    tables = [x]
    for (wg, bg, wb, bb) in ((W_gc_0, b_gc_0, W_bi_0, b_bi_0),
                             (W_gc_1, b_gc_1, W_bi_1, b_bi_1)):
        acc = _spmm_kernel(x, rowp, colp, zeros32)
        ego, x = _dense_call(acc, ego, deg, wg, bg, wb, bb, p_mat)
        tables.append(x)
    acc3 = _spmm_kernel(x, rowp, colp, zeros32)

    bidx = jnp.concatenate([users.astype(jnp.int32),
                            pos_items.astype(jnp.int32) + N_USER,
                            neg_items.astype(jnp.int32) + N_USER])
    x0r, x1r, x2r, acc3r, degr = _bgather_kernel(
        tables[0], tables[1], tables[2], acc3, deg, bidx)
    u_g, p_g, n_g = _finish_call(x0r, x1r, x2r, acc3r, degr,
                                 W_gc_2, b_gc_2, W_bi_2, b_bi_2, pt_mat)
    return (u_g, p_g, n_g)


# speedup vs baseline: 1.3440x; 1.3440x over previous
import functools

import jax
import jax.numpy as jnp
from jax import lax
from jax.experimental import pallas as pl
from jax.experimental.pallas import tpu as pltpu
from jax.experimental.pallas import tpu_sc as plsc

N_USER = 60000
N_ITEM = 40000
N = N_USER + N_ITEM
E_PAIRS = 800000
D = 32
BATCH = 1024

NC = 2
NS = 16
L = 16

CHUNK = 128
NBUF = 2
NGROUPS = 196
NBUF_H = 4
NGROUPS_H = 98
CHUNKS_PER_SUB = NBUF * NGROUPS
EDGES_PER_SUB = CHUNKS_PER_SUB * CHUNK
HALF_PAD = EDGES_PER_SUB * NS
ACC_ROWS = 60032
DUMP_ROW = 60000
ZROWS_PER_SUB = ACC_ROWS // NS
U_SPAN = 3752
U_LAST = N_USER - 15 * U_SPAN
I_SPAN = 2504
I_LAST = N_ITEM - 15 * I_SPAN
HIST_W = 16

_mesh = plsc.VectorSubcoreMesh(
    core_axis_name="c", subcore_axis_name="s", num_cores=NC, num_subcores=NS)
_sc_params = pltpu.CompilerParams(use_tc_tiling_on_sc=False)


def _writeout(acc_sh, out, c, s):
    @pl.when(jnp.logical_and(c == 0, s < 15))
    def _():
        pltpu.sync_copy(acc_sh.at[pl.ds(s * U_SPAN, U_SPAN)],
                        out.at[pl.ds(s * U_SPAN, U_SPAN)])

    @pl.when(jnp.logical_and(c == 0, s == 15))
    def _():
        pltpu.sync_copy(acc_sh.at[pl.ds(15 * U_SPAN, U_LAST)],
                        out.at[pl.ds(15 * U_SPAN, U_LAST)])

    @pl.when(jnp.logical_and(c == 1, s < 15))
    def _():
        pltpu.sync_copy(acc_sh.at[pl.ds(s * I_SPAN, I_SPAN)],
                        out.at[pl.ds(N_USER + s * I_SPAN, I_SPAN)])

    @pl.when(jnp.logical_and(c == 1, s == 15))
    def _():
        pltpu.sync_copy(acc_sh.at[pl.ds(15 * I_SPAN, I_LAST)],
                        out.at[pl.ds(N_USER + 15 * I_SPAN, I_LAST)])


def _hist_body(rowp, zeros16, deg, acc_sh, basis, *rest):
    ridx = rest[0:NBUF_H]
    sem_i = rest[NBUF_H:2 * NBUF_H]
    sem_s = rest[2 * NBUF_H:3 * NBUF_H]
    c = lax.axis_index("c")
    s = lax.axis_index("s")
    pltpu.sync_copy(zeros16.at[pl.ds(s * ZROWS_PER_SUB, ZROWS_PER_SUB)],
                    acc_sh.at[pl.ds(s * ZROWS_PER_SUB, ZROWS_PER_SUB)])
    e0 = jnp.where(lax.iota(jnp.int32, L) == 0,
                   jnp.float32(1), jnp.float32(0))

    @pl.loop(0, CHUNK)
    def _(i):
        basis[i, :] = e0

    plsc.subcore_barrier()
    base = c * HALF_PAD + s * EDGES_PER_SUB

    @pl.loop(0, NGROUPS_H)
    def _(grp):
        cbase = base + grp * (NBUF_H * CHUNK)
        descs = []
        for b in range(NBUF_H):
            @pl.when(grp > 0)
            def _(b=b):
                pltpu.make_async_copy(
                    zeros16.at[pl.ds(0, CHUNK)], basis, sem_s[b]).wait()
            descs.append(pltpu.async_copy(
                rowp.at[pl.ds(cbase + b * CHUNK, CHUNK)], ridx[b], sem_i[b]))
        for b in range(NBUF_H):
            descs[b].wait()
            pltpu.async_copy(basis, acc_sh.at[ridx[b]], sem_s[b], add=True)

    for b in range(NBUF_H):
        pltpu.make_async_copy(
            zeros16.at[pl.ds(0, CHUNK)], basis, sem_s[b]).wait()

    plsc.subcore_barrier()
    _writeout(acc_sh, deg, c, s)


_hist_kernel = pl.kernel(
    _hist_body,
    out_type=jax.ShapeDtypeStruct((N, HIST_W), jnp.float32),
    mesh=_mesh,
    scratch_types=(
        [pltpu.VMEM_SHARED((ACC_ROWS, HIST_W), jnp.float32)]
        + [pltpu.VMEM((CHUNK, HIST_W), jnp.float32)]
        + [pltpu.VMEM((CHUNK,), jnp.int32)] * NBUF_H
        + [pltpu.SemaphoreType.DMA] * (2 * NBUF_H)
    ),
    compiler_params=_sc_params,
)


def _spmm_body(x, eidx, zeros32, acc, acc_sh, *rest):
    ibuf = rest[0:NBUF]
    rbuf = rest[NBUF:2 * NBUF]
    sem_i = rest[2 * NBUF:3 * NBUF]
    sem_g = rest[3 * NBUF:4 * NBUF]
    sem_s = rest[4 * NBUF:5 * NBUF]
    c = lax.axis_index("c")
    s = lax.axis_index("s")
    pltpu.sync_copy(zeros32.at[pl.ds(s * ZROWS_PER_SUB, ZROWS_PER_SUB)],
                    acc_sh.at[pl.ds(s * ZROWS_PER_SUB, ZROWS_PER_SUB)])
    plsc.subcore_barrier()
    cbase0 = (c * HALF_PAD + s * EDGES_PER_SUB) // CHUNK

    @pl.loop(0, NGROUPS)
    def _(grp):
        cbase = cbase0 + grp * NBUF
        descs = []
        for b in range(NBUF):
            @pl.when(grp > 0)
            def _(b=b):
                pltpu.make_async_copy(
                    x.at[pl.ds(0, CHUNK)], rbuf[b], sem_s[b]).wait()
            descs.append(pltpu.async_copy(
                eidx.at[cbase + b], ibuf[b], sem_i[b]))
        for b in range(NBUF):
            descs[b].wait()
            pltpu.async_copy(x.at[ibuf[b].at[1]], rbuf[b], sem_g[b])
        for b in range(NBUF):
            pltpu.make_async_copy(
                x.at[pl.ds(0, CHUNK)], rbuf[b], sem_g[b]).wait()
            pltpu.async_copy(rbuf[b], acc_sh.at[ibuf[b].at[0]],
                             sem_s[b], add=True)

    for b in range(NBUF):
        pltpu.make_async_copy(x.at[pl.ds(0, CHUNK)], rbuf[b], sem_s[b]).wait()

    plsc.subcore_barrier()
    _writeout(acc_sh, acc, c, s)


_spmm_kernel = pl.kernel(
    _spmm_body,
    out_type=jax.ShapeDtypeStruct((N, D), jnp.float32),
    mesh=_mesh,
    scratch_types=(
        [pltpu.VMEM_SHARED((ACC_ROWS, D), jnp.float32)]
        + [pltpu.VMEM((2, CHUNK), jnp.int32)] * NBUF
        + [pltpu.VMEM((CHUNK, D), jnp.float32)] * NBUF
        + [pltpu.SemaphoreType.DMA] * (3 * NBUF)
    ),
    compiler_params=_sc_params,
)

ROWS_PER_GW = BATCH // (NC * NS)


def _bgather_body(t0, t1, t2, t3, deg, bidx, o0, o1, o2, o3, odeg,
                  ibuf, rbuf, dbuf, dma):
    c = lax.axis_index("c")
    s = lax.axis_index("s")
    w = s * NC + c
    base = w * (3 * BATCH // (NC * NS))
    span = 3 * BATCH // (NC * NS)
    pltpu.sync_copy(bidx.at[pl.ds(base, span)], ibuf)
    for t, out in ((t0, o0), (t1, o1), (t2, o2), (t3, o3)):
        pltpu.async_copy(t.at[ibuf], rbuf, dma).wait()
        pltpu.sync_copy(rbuf, out.at[pl.ds(base, span)])
    pltpu.async_copy(deg.at[ibuf], dbuf, dma).wait()
    pltpu.sync_copy(dbuf, odeg.at[pl.ds(base, span)])


_bgather_kernel = pl.kernel(
    _bgather_body,
    out_type=[jax.ShapeDtypeStruct((3 * BATCH, D), jnp.float32)] * 4
    + [jax.ShapeDtypeStruct((3 * BATCH, HIST_W), jnp.float32)],
    mesh=_mesh,
    scratch_types=[
        pltpu.VMEM((3 * BATCH // (NC * NS),), jnp.int32),
        pltpu.VMEM((3 * BATCH // (NC * NS), D), jnp.float32),
        pltpu.VMEM((3 * BATCH // (NC * NS), HIST_W), jnp.float32),
        pltpu.SemaphoreType.DMA,
    ],
    compiler_params=_sc_params,
)


BR = 5000
GRID = N // BR


def _prep_body(deg_ref, ego_ref, x_ref):
    dinv = lax.rsqrt(jnp.maximum(deg_ref[:, :1], 1.0))
    x_ref[...] = ego_ref[...] * dinv


_prep_call = pl.pallas_call(
    _prep_body,
    grid=(GRID,),
    in_specs=[
        pl.BlockSpec((BR, HIST_W), lambda i: (i, 0)),
        pl.BlockSpec((BR, D), lambda i: (i, 0)),
    ],
    out_specs=pl.BlockSpec((BR, D), lambda i: (i, 0)),
    out_shape=jax.ShapeDtypeStruct((N, D), jnp.float32),
)


def _dense_body(acc_ref, ego_ref, deg_ref, wg_ref, bg_ref, wb_ref, bb_ref,
                h_ref, xn_ref):
    dinv = lax.rsqrt(jnp.maximum(deg_ref[:, :1], 1.0))
    side = acc_ref[...] * dinv
    s_emb = jnp.dot(side, wg_ref[...],
                    preferred_element_type=jnp.float32) + bg_ref[...]
    b_emb = jnp.dot(ego_ref[...] * side, wb_ref[...],
                    preferred_element_type=jnp.float32) + bb_ref[...]
    z = s_emb + b_emb
    h = jnp.where(z >= 0, z, 0.2 * z)
    h_ref[...] = h
    xn_ref[...] = h * dinv


_dense_call = pl.pallas_call(
    _dense_body,
    grid=(GRID,),
    in_specs=[
        pl.BlockSpec((BR, D), lambda i: (i, 0)),
        pl.BlockSpec((BR, D), lambda i: (i, 0)),
        pl.BlockSpec((BR, HIST_W), lambda i: (i, 0)),
        pl.BlockSpec((D, D), lambda i: (0, 0)),
        pl.BlockSpec((1, D), lambda i: (0, 0)),
        pl.BlockSpec((D, D), lambda i: (0, 0)),
        pl.BlockSpec((1, D), lambda i: (0, 0)),
    ],
    out_specs=[pl.BlockSpec((BR, D), lambda i: (i, 0))] * 2,
    out_shape=[jax.ShapeDtypeStruct((N, D), jnp.float32)] * 2,
)


def _finish_body(x0r, x1r, x2r, acc3r, degr, wg_ref, bg_ref, wb_ref, bb_ref,
                 ug, pg, ng):
    dall = jnp.maximum(degr[:, :1], 1.0)
    scale_all = jnp.sqrt(dall)
    dinv_all = lax.rsqrt(dall)
    side3 = acc3r[...] * dinv_all
    ego2r = x2r[...] * scale_all
    z = (jnp.dot(side3, wg_ref[...], preferred_element_type=jnp.float32)
         + bg_ref[...]
         + jnp.dot(ego2r * side3, wb_ref[...],
                   preferred_element_type=jnp.float32) + bb_ref[...])
    h3 = jnp.where(z >= 0, z, 0.2 * z)
    for b, out in enumerate((ug, pg, ng)):
        sl = slice(b * BATCH, (b + 1) * BATCH)
        out[:, 0:D] = x0r[sl, :] * scale_all[sl, :]
        for t, v in enumerate((x1r[sl, :], x2r[sl, :], h3[sl, :])):
            nrm = jnp.maximum(
                jnp.sqrt(jnp.sum(v * v, axis=1, keepdims=True)), 1e-12)
            out[:, (t + 1) * D:(t + 2) * D] = v / nrm


_finish_call = pl.pallas_call(
    _finish_body,
    grid=(1,),
    in_specs=[pl.BlockSpec((3 * BATCH, D), lambda i: (0, 0))] * 4
    + [pl.BlockSpec((3 * BATCH, HIST_W), lambda i: (0, 0))]
    + [pl.BlockSpec((D, D), lambda i: (0, 0)),
       pl.BlockSpec((1, D), lambda i: (0, 0)),
       pl.BlockSpec((D, D), lambda i: (0, 0)),
       pl.BlockSpec((1, D), lambda i: (0, 0))],
    out_specs=[pl.BlockSpec((BATCH, 4 * D), lambda i: (0, 0))] * 3,
    out_shape=[jax.ShapeDtypeStruct((BATCH, 4 * D), jnp.float32)] * 3,
)


def kernel(users, pos_items, neg_items, edge_index, adj_values, user_emb,
           item_emb, W_gc_0, b_gc_0, W_bi_0, b_bi_0, W_gc_1, b_gc_1, W_bi_1,
           b_bi_1, W_gc_2, b_gc_2, W_bi_2, b_bi_2):
    del adj_values
    row = edge_index[0].astype(jnp.int32)
    col = edge_index[1].astype(jnp.int32)
    pad_n = HALF_PAD - E_PAIRS
    pad_row = jnp.full((pad_n,), DUMP_ROW, jnp.int32)
    pad_col = jnp.zeros((pad_n,), jnp.int32)
    rowp = jnp.concatenate(
        [row[:E_PAIRS], pad_row, row[E_PAIRS:] - N_USER, pad_row])
    colp = jnp.concatenate([col[:E_PAIRS], pad_col, col[E_PAIRS:], pad_col])
    eidx = jnp.stack([rowp.reshape(-1, CHUNK), colp.reshape(-1, CHUNK)],
                     axis=1)

    zeros16 = jnp.zeros((ACC_ROWS, HIST_W), jnp.float32)
    zeros32 = jnp.zeros((ACC_ROWS, D), jnp.float32)

    ego = jnp.concatenate([user_emb, item_emb], axis=0)
    deg = _hist_kernel(rowp, zeros16)
    x = _prep_call(deg, ego)

    tables = [x]
    for (wg, bg, wb, bb) in ((W_gc_0, b_gc_0, W_bi_0, b_bi_0),
                             (W_gc_1, b_gc_1, W_bi_1, b_bi_1)):
        acc = _spmm_kernel(x, eidx, zeros32)
        ego, x = _dense_call(acc, ego, deg, wg, bg, wb, bb)
        tables.append(x)
    acc3 = _spmm_kernel(x, eidx, zeros32)

    bidx = jnp.concatenate([users.astype(jnp.int32),
                            pos_items.astype(jnp.int32) + N_USER,
                            neg_items.astype(jnp.int32) + N_USER])
    x0r, x1r, x2r, acc3r, degr = _bgather_kernel(
        tables[0], tables[1], tables[2], acc3, deg, bidx)
    u_g, p_g, n_g = _finish_call(x0r, x1r, x2r, acc3r, degr,
                                 W_gc_2, b_gc_2, W_bi_2, b_bi_2)
    return (u_g, p_g, n_g)


# speedup vs baseline: 1.3494x; 1.0040x over previous
import functools

import jax
import jax.numpy as jnp
from jax import lax
from jax.experimental import pallas as pl
from jax.experimental.pallas import tpu as pltpu
from jax.experimental.pallas import tpu_sc as plsc

N_USER = 60000
N_ITEM = 40000
N = N_USER + N_ITEM
E_PAIRS = 800000
D = 32
BATCH = 1024

NC = 2
NS = 16
L = 16

CHUNK = 128
NBUF = 2
NGROUPS = 196
NBUF_H = 4
NGROUPS_H = 98
CHUNKS_PER_SUB = NBUF * NGROUPS
EDGES_PER_SUB = CHUNKS_PER_SUB * CHUNK
HALF_PAD = EDGES_PER_SUB * NS
ACC_ROWS = 60032
DUMP_ROW = 60000
ZROWS_PER_SUB = ACC_ROWS // NS
U_SPAN = 3752
U_LAST = N_USER - 15 * U_SPAN
I_SPAN = 2504
I_LAST = N_ITEM - 15 * I_SPAN
HIST_W = 16

_mesh = plsc.VectorSubcoreMesh(
    core_axis_name="c", subcore_axis_name="s", num_cores=NC, num_subcores=NS)
_sc_params = pltpu.CompilerParams(use_tc_tiling_on_sc=False)


def _writeout(acc_sh, out, c, s):
    @pl.when(jnp.logical_and(c == 0, s < 15))
    def _():
        pltpu.sync_copy(acc_sh.at[pl.ds(s * U_SPAN, U_SPAN)],
                        out.at[pl.ds(s * U_SPAN, U_SPAN)])

    @pl.when(jnp.logical_and(c == 0, s == 15))
    def _():
        pltpu.sync_copy(acc_sh.at[pl.ds(15 * U_SPAN, U_LAST)],
                        out.at[pl.ds(15 * U_SPAN, U_LAST)])

    @pl.when(jnp.logical_and(c == 1, s < 15))
    def _():
        pltpu.sync_copy(acc_sh.at[pl.ds(s * I_SPAN, I_SPAN)],
                        out.at[pl.ds(N_USER + s * I_SPAN, I_SPAN)])

    @pl.when(jnp.logical_and(c == 1, s == 15))
    def _():
        pltpu.sync_copy(acc_sh.at[pl.ds(15 * I_SPAN, I_LAST)],
                        out.at[pl.ds(N_USER + 15 * I_SPAN, I_LAST)])


def _hist_body(rowp, zeros16, deg, acc_sh, basis, *rest):
    ridx = rest[0:NBUF_H]
    sem_i = rest[NBUF_H:2 * NBUF_H]
    sem_s = rest[2 * NBUF_H:3 * NBUF_H]
    c = lax.axis_index("c")
    s = lax.axis_index("s")
    pltpu.sync_copy(zeros16.at[pl.ds(s * ZROWS_PER_SUB, ZROWS_PER_SUB)],
                    acc_sh.at[pl.ds(s * ZROWS_PER_SUB, ZROWS_PER_SUB)])
    e0 = jnp.where(lax.iota(jnp.int32, L) == 0,
                   jnp.float32(1), jnp.float32(0))

    @pl.loop(0, CHUNK)
    def _(i):
        basis[i, :] = e0

    plsc.subcore_barrier()
    base = c * HALF_PAD + s * EDGES_PER_SUB

    @pl.loop(0, NGROUPS_H)
    def _(grp):
        cbase = base + grp * (NBUF_H * CHUNK)
        descs = []
        for b in range(NBUF_H):
            @pl.when(grp > 0)
            def _(b=b):
                pltpu.make_async_copy(
                    zeros16.at[pl.ds(0, CHUNK)], basis, sem_s[b]).wait()
            descs.append(pltpu.async_copy(
                rowp.at[pl.ds(cbase + b * CHUNK, CHUNK)], ridx[b], sem_i[b]))
        for b in range(NBUF_H):
            descs[b].wait()
            pltpu.async_copy(basis, acc_sh.at[ridx[b]], sem_s[b], add=True)

    for b in range(NBUF_H):
        pltpu.make_async_copy(
            zeros16.at[pl.ds(0, CHUNK)], basis, sem_s[b]).wait()

    plsc.subcore_barrier()
    _writeout(acc_sh, deg, c, s)


_hist_kernel = pl.kernel(
    _hist_body,
    out_type=jax.ShapeDtypeStruct((N, HIST_W), jnp.float32),
    mesh=_mesh,
    scratch_types=(
        [pltpu.VMEM_SHARED((ACC_ROWS, HIST_W), jnp.float32)]
        + [pltpu.VMEM((CHUNK, HIST_W), jnp.float32)]
        + [pltpu.VMEM((CHUNK,), jnp.int32)] * NBUF_H
        + [pltpu.SemaphoreType.DMA] * (2 * NBUF_H)
    ),
    compiler_params=_sc_params,
)


def _spmm_core(x, eidx, zeros32, acc_sh, ibuf, rbuf, sem_i, sem_g, sem_s,
               c, s):
    pltpu.sync_copy(zeros32.at[pl.ds(s * ZROWS_PER_SUB, ZROWS_PER_SUB)],
                    acc_sh.at[pl.ds(s * ZROWS_PER_SUB, ZROWS_PER_SUB)])
    plsc.subcore_barrier()
    cbase0 = (c * HALF_PAD + s * EDGES_PER_SUB) // CHUNK

    @pl.loop(0, NGROUPS)
    def _(grp):
        cbase = cbase0 + grp * NBUF
        descs = []
        for b in range(NBUF):
            @pl.when(grp > 0)
            def _(b=b):
                pltpu.make_async_copy(
                    x.at[pl.ds(0, CHUNK)], rbuf[b], sem_s[b]).wait()
            descs.append(pltpu.async_copy(
                eidx.at[cbase + b], ibuf[b], sem_i[b]))
        for b in range(NBUF):
            descs[b].wait()
            pltpu.async_copy(x.at[ibuf[b].at[1]], rbuf[b], sem_g[b])
        for b in range(NBUF):
            pltpu.make_async_copy(
                x.at[pl.ds(0, CHUNK)], rbuf[b], sem_g[b]).wait()
            pltpu.async_copy(rbuf[b], acc_sh.at[ibuf[b].at[0]],
                             sem_s[b], add=True)

    for b in range(NBUF):
        pltpu.make_async_copy(x.at[pl.ds(0, CHUNK)], rbuf[b], sem_s[b]).wait()


def _spmm_body(x, eidx, zeros32, acc, acc_sh, *rest):
    ibuf = rest[0:NBUF]
    rbuf = rest[NBUF:2 * NBUF]
    sem_i = rest[2 * NBUF:3 * NBUF]
    sem_g = rest[3 * NBUF:4 * NBUF]
    sem_s = rest[4 * NBUF:5 * NBUF]
    c = lax.axis_index("c")
    s = lax.axis_index("s")
    _spmm_core(x, eidx, zeros32, acc_sh, ibuf, rbuf, sem_i, sem_g, sem_s,
               c, s)
    plsc.subcore_barrier()
    _writeout(acc_sh, acc, c, s)


_spmm_kernel = pl.kernel(
    _spmm_body,
    out_type=jax.ShapeDtypeStruct((N, D), jnp.float32),
    mesh=_mesh,
    scratch_types=(
        [pltpu.VMEM_SHARED((ACC_ROWS, D), jnp.float32)]
        + [pltpu.VMEM((2, CHUNK), jnp.int32)] * NBUF
        + [pltpu.VMEM((CHUNK, D), jnp.float32)] * NBUF
        + [pltpu.SemaphoreType.DMA] * (3 * NBUF)
    ),
    compiler_params=_sc_params,
)

GROWS = 3 * BATCH // (NC * NS)


def _spmm3_body(x, eidx, zeros32, x0, x1, deg, bidxg, bidxl,
                o0, o1, o2, odeg, oaccu, oaccpn, acc_sh, *rest):
    ibuf = rest[0:NBUF]
    rbuf = rest[NBUF:2 * NBUF]
    sem_i = rest[2 * NBUF:3 * NBUF]
    sem_g = rest[3 * NBUF:4 * NBUF]
    sem_s = rest[4 * NBUF:5 * NBUF]
    gib, gibu, gibp, gdb = rest[5 * NBUF:5 * NBUF + 4]
    c = lax.axis_index("c")
    s = lax.axis_index("s")
    _spmm_core(x, eidx, zeros32, acc_sh, ibuf, rbuf, sem_i, sem_g, sem_s,
               c, s)
    w = s * NC + c
    gb = w * GROWS
    pltpu.sync_copy(bidxg.at[pl.ds(gb, GROWS)], gib)
    for t, out in ((x0, o0), (x1, o1), (x, o2)):
        pltpu.async_copy(t.at[gib], rbuf[0].at[pl.ds(0, GROWS)],
                         sem_g[0]).wait()
        pltpu.sync_copy(rbuf[0].at[pl.ds(0, GROWS)], out.at[pl.ds(gb, GROWS)])
    pltpu.async_copy(deg.at[gib], gdb, sem_g[0]).wait()
    pltpu.sync_copy(gdb, odeg.at[pl.ds(gb, GROWS)])
    plsc.subcore_barrier()

    @pl.when(c == 0)
    def _():
        pltpu.sync_copy(bidxl.at[pl.ds(s * 64, 64)], gibu)
        pltpu.async_copy(acc_sh.at[gibu], rbuf[0].at[pl.ds(0, 64)],
                         sem_g[0]).wait()
        pltpu.sync_copy(rbuf[0].at[pl.ds(0, 64)], oaccu.at[pl.ds(s * 64, 64)])

    @pl.when(c == 1)
    def _():
        pltpu.sync_copy(bidxl.at[pl.ds(BATCH + s * 128, 128)], gibp)
        pltpu.async_copy(acc_sh.at[gibp], rbuf[1], sem_g[0]).wait()
        pltpu.sync_copy(rbuf[1], oaccpn.at[pl.ds(s * 128, 128)])


_spmm3_kernel = pl.kernel(
    _spmm3_body,
    out_type=[jax.ShapeDtypeStruct((3 * BATCH, D), jnp.float32)] * 3
    + [jax.ShapeDtypeStruct((3 * BATCH, HIST_W), jnp.float32)]
    + [jax.ShapeDtypeStruct((BATCH, D), jnp.float32),
       jax.ShapeDtypeStruct((2 * BATCH, D), jnp.float32)],
    mesh=_mesh,
    scratch_types=(
        [pltpu.VMEM_SHARED((ACC_ROWS, D), jnp.float32)]
        + [pltpu.VMEM((2, CHUNK), jnp.int32)] * NBUF
        + [pltpu.VMEM((CHUNK, D), jnp.float32)] * NBUF
        + [pltpu.SemaphoreType.DMA] * (3 * NBUF)
        + [pltpu.VMEM((GROWS,), jnp.int32),
           pltpu.VMEM((64,), jnp.int32),
           pltpu.VMEM((128,), jnp.int32),
           pltpu.VMEM((GROWS, HIST_W), jnp.float32)]
    ),
    compiler_params=_sc_params,
)

ROWS_PER_GW = BATCH // (NC * NS)


def _bgather_body(t0, t1, t2, t3, deg, bidx, o0, o1, o2, o3, odeg,
                  ibuf, rbuf, dbuf, dma):
    c = lax.axis_index("c")
    s = lax.axis_index("s")
    w = s * NC + c
    base = w * (3 * BATCH // (NC * NS))
    span = 3 * BATCH // (NC * NS)
    pltpu.sync_copy(bidx.at[pl.ds(base, span)], ibuf)
    for t, out in ((t0, o0), (t1, o1), (t2, o2), (t3, o3)):
        pltpu.async_copy(t.at[ibuf], rbuf, dma).wait()
        pltpu.sync_copy(rbuf, out.at[pl.ds(base, span)])
    pltpu.async_copy(deg.at[ibuf], dbuf, dma).wait()
    pltpu.sync_copy(dbuf, odeg.at[pl.ds(base, span)])


_bgather_kernel = pl.kernel(
    _bgather_body,
    out_type=[jax.ShapeDtypeStruct((3 * BATCH, D), jnp.float32)] * 4
    + [jax.ShapeDtypeStruct((3 * BATCH, HIST_W), jnp.float32)],
    mesh=_mesh,
    scratch_types=[
        pltpu.VMEM((3 * BATCH // (NC * NS),), jnp.int32),
        pltpu.VMEM((3 * BATCH // (NC * NS), D), jnp.float32),
        pltpu.VMEM((3 * BATCH // (NC * NS), HIST_W), jnp.float32),
        pltpu.SemaphoreType.DMA,
    ],
    compiler_params=_sc_params,
)


BR = 5000
GRID = N // BR


def _prep_body(deg_ref, ego_ref, x_ref):
    dinv = lax.rsqrt(jnp.maximum(deg_ref[:, :1], 1.0))
    x_ref[...] = ego_ref[...] * dinv


_prep_call = pl.pallas_call(
    _prep_body,
    grid=(GRID,),
    in_specs=[
        pl.BlockSpec((BR, HIST_W), lambda i: (i, 0)),
        pl.BlockSpec((BR, D), lambda i: (i, 0)),
    ],
    out_specs=pl.BlockSpec((BR, D), lambda i: (i, 0)),
    out_shape=jax.ShapeDtypeStruct((N, D), jnp.float32),
)


def _dense_body(acc_ref, ego_ref, deg_ref, wg_ref, bg_ref, wb_ref, bb_ref,
                h_ref, xn_ref):
    dinv = lax.rsqrt(jnp.maximum(deg_ref[:, :1], 1.0))
    side = acc_ref[...] * dinv
    s_emb = jnp.dot(side, wg_ref[...],
                    preferred_element_type=jnp.float32) + bg_ref[...]
    b_emb = jnp.dot(ego_ref[...] * side, wb_ref[...],
                    preferred_element_type=jnp.float32) + bb_ref[...]
    z = s_emb + b_emb
    h = jnp.where(z >= 0, z, 0.2 * z)
    h_ref[...] = h
    xn_ref[...] = h * dinv


_dense_call = pl.pallas_call(
    _dense_body,
    grid=(GRID,),
    in_specs=[
        pl.BlockSpec((BR, D), lambda i: (i, 0)),
        pl.BlockSpec((BR, D), lambda i: (i, 0)),
        pl.BlockSpec((BR, HIST_W), lambda i: (i, 0)),
        pl.BlockSpec((D, D), lambda i: (0, 0)),
        pl.BlockSpec((1, D), lambda i: (0, 0)),
        pl.BlockSpec((D, D), lambda i: (0, 0)),
        pl.BlockSpec((1, D), lambda i: (0, 0)),
    ],
    out_specs=[pl.BlockSpec((BR, D), lambda i: (i, 0))] * 2,
    out_shape=[jax.ShapeDtypeStruct((N, D), jnp.float32)] * 2,
)


def _finish_body(x0r, x1r, x2r, acc3r, degr, wg_ref, bg_ref, wb_ref, bb_ref,
                 ug, pg, ng):
    dall = jnp.maximum(degr[:, :1], 1.0)
    scale_all = jnp.sqrt(dall)
    dinv_all = lax.rsqrt(dall)
    side3 = acc3r[...] * dinv_all
    ego2r = x2r[...] * scale_all
    z = (jnp.dot(side3, wg_ref[...], preferred_element_type=jnp.float32)
         + bg_ref[...]
         + jnp.dot(ego2r * side3, wb_ref[...],
                   preferred_element_type=jnp.float32) + bb_ref[...])
    h3 = jnp.where(z >= 0, z, 0.2 * z)
    for b, out in enumerate((ug, pg, ng)):
        sl = slice(b * BATCH, (b + 1) * BATCH)
        out[:, 0:D] = x0r[sl, :] * scale_all[sl, :]
        for t, v in enumerate((x1r[sl, :], x2r[sl, :], h3[sl, :])):
            nrm = jnp.maximum(
                jnp.sqrt(jnp.sum(v * v, axis=1, keepdims=True)), 1e-12)
            out[:, (t + 1) * D:(t + 2) * D] = v / nrm


_finish_call = pl.pallas_call(
    _finish_body,
    grid=(1,),
    in_specs=[pl.BlockSpec((3 * BATCH, D), lambda i: (0, 0))] * 4
    + [pl.BlockSpec((3 * BATCH, HIST_W), lambda i: (0, 0))]
    + [pl.BlockSpec((D, D), lambda i: (0, 0)),
       pl.BlockSpec((1, D), lambda i: (0, 0)),
       pl.BlockSpec((D, D), lambda i: (0, 0)),
       pl.BlockSpec((1, D), lambda i: (0, 0))],
    out_specs=[pl.BlockSpec((BATCH, 4 * D), lambda i: (0, 0))] * 3,
    out_shape=[jax.ShapeDtypeStruct((BATCH, 4 * D), jnp.float32)] * 3,
)


def kernel(users, pos_items, neg_items, edge_index, adj_values, user_emb,
           item_emb, W_gc_0, b_gc_0, W_bi_0, b_bi_0, W_gc_1, b_gc_1, W_bi_1,
           b_bi_1, W_gc_2, b_gc_2, W_bi_2, b_bi_2):
    del adj_values
    row = edge_index[0].astype(jnp.int32)
    col = edge_index[1].astype(jnp.int32)
    pad_n = HALF_PAD - E_PAIRS
    pad_row = jnp.full((pad_n,), DUMP_ROW, jnp.int32)
    pad_col = jnp.zeros((pad_n,), jnp.int32)
    rowp = jnp.concatenate(
        [row[:E_PAIRS], pad_row, row[E_PAIRS:] - N_USER, pad_row])
    colp = jnp.concatenate([col[:E_PAIRS], pad_col, col[E_PAIRS:], pad_col])
    eidx = jnp.stack([rowp.reshape(-1, CHUNK), colp.reshape(-1, CHUNK)],
                     axis=1)

    zeros16 = jnp.zeros((ACC_ROWS, HIST_W), jnp.float32)
    zeros32 = jnp.zeros((ACC_ROWS, D), jnp.float32)

    ego = jnp.concatenate([user_emb, item_emb], axis=0)
    deg = _hist_kernel(rowp, zeros16)
    x = _prep_call(deg, ego)

    tables = [x]
    for (wg, bg, wb, bb) in ((W_gc_0, b_gc_0, W_bi_0, b_bi_0),
                             (W_gc_1, b_gc_1, W_bi_1, b_bi_1)):
        acc = _spmm_kernel(x, eidx, zeros32)
        ego, x = _dense_call(acc, ego, deg, wg, bg, wb, bb)
        tables.append(x)
    bidxg = jnp.concatenate([users.astype(jnp.int32),
                             pos_items.astype(jnp.int32) + N_USER,
                             neg_items.astype(jnp.int32) + N_USER])
    bidxl = jnp.concatenate([users.astype(jnp.int32),
                             pos_items.astype(jnp.int32),
                             neg_items.astype(jnp.int32)])
    x0r, x1r, x2r, degr, accu, accpn = _spmm3_kernel(
        x, eidx, zeros32, tables[0], tables[1], deg, bidxg, bidxl)
    acc3r = jnp.concatenate([accu, accpn])
    u_g, p_g, n_g = _finish_call(x0r, x1r, x2r, acc3r, degr,
                                 W_gc_2, b_gc_2, W_bi_2, b_bi_2)
    return (u_g, p_g, n_g)
